# baseline (speedup 1.0000x reference)
import functools

import jax
import jax.numpy as jnp
from jax import lax
from jax.experimental import pallas as pl
from jax.experimental.pallas import tpu as pltpu
from jax.experimental.pallas import tpu_sc as plsc

N, E, D, H, O = 10000, 320000, 128, 128, 128

NC, NS = 2, 16
CHUNK = 128
NCHUNK = 158
EPT = NCHUNK * CHUNK
EPAD = NS * EPT
IDXLEN = NC * EPAD + CHUNK
NROWS = 10112
ZPT = NROWS // NS

_ROWBLK = 1000


def _ln(h, g, b):
    m = jnp.mean(h, axis=-1, keepdims=True)
    c = h - m
    v = jnp.mean(c * c, axis=-1, keepdims=True)
    return c / jnp.sqrt(v + 1e-5) * g + b


def _dot(a, b):
    return jax.lax.dot_general(a, b, (((1,), (0,)), ((), ())),
                               preferred_element_type=jnp.float32,
                               precision=jax.lax.Precision.HIGHEST)


def _gelu(x):
    return 0.5 * x * (1.0 + lax.erf(x * 0.7071067811865476))


def _branch(x1, W1, b1, W2, b2, g, b):
    h = _gelu(_dot(x1, W1[...]) + b1[...])
    h = _dot(h, W2[...]) + b2[...]
    return _ln(h, g[...], b[...])


def _tc1a_body(x_ref, linW, linb, rtW1, rtb1, rtW2, rtb2, rtg, rtb,
               tab_o, x1_o):
    x = x_ref[...]
    x1 = _dot(x, linW[...]) + linb[...]
    beta = _branch(x1, rtW1, rtb1, rtW2, rtb2, rtg, rtb)
    tab_o[0, :, :] = x1
    tab_o[1, :, :] = beta * x1
    x1_o[...] = x1


def _tc1b_body(x1_ref, dbW1, dbb1, dbW2, dbb2, dbg, dbb,
               rbW1, rbb1, rbW2, rbb2, rbg, rbb, ng, nb,
               alpha_o, gamma_o, xres_o):
    x1 = x1_ref[...]
    xres_o[...] = _ln(x1, ng[...], nb[...])
    alpha_o[...] = _branch(x1, dbW1, dbb1, dbW2, dbb2, dbg, dbb)
    gamma_o[...] = _branch(x1, rbW1, rbb1, rbW2, rbb2, rbg, rbb)


def _tc2_body(seg_ref, alpha_ref, gamma_ref, xres_ref,
              fcW1, fcb1, fcW2, fcb2, out_o):
    x2 = alpha_ref[...] * seg_ref[0, :, :] + gamma_ref[...] + seg_ref[1, :, :]
    h = _gelu(_dot(x2, fcW1[...]) + fcb1[...])
    out_o[...] = _dot(h, fcW2[...]) + fcb2[...] + xres_ref[...]


def _row_spec():
    return pl.BlockSpec((_ROWBLK, H), lambda i: (i, 0))


def _full_spec(shape):
    return pl.BlockSpec(shape, lambda i: tuple(0 for _ in shape))


def _tc1a(x, p):
    grid = N // _ROWBLK
    w = _full_spec((H, H))
    b = _full_spec((1, H))
    return pl.pallas_call(
        _tc1a_body,
        grid=(grid,),
        in_specs=[_row_spec(), w, b, w, b, w, b, b, b],
        out_specs=[pl.BlockSpec((2, _ROWBLK, H), lambda i: (0, i, 0)),
                   _row_spec()],
        out_shape=[jax.ShapeDtypeStruct((2, N, H), jnp.float32),
                   jax.ShapeDtypeStruct((N, H), jnp.float32)],
    )(x,
      p['lin_W'].T, p['lin_b'][None],
      p['rt_W1'].T, p['rt_b1'][None], p['rt_W2'].T, p['rt_b2'][None],
      p['rt_g'][None], p['rt_b'][None])


def _tc1b(x1, p):
    grid = N // _ROWBLK
    w = _full_spec((H, H))
    b = _full_spec((1, H))
    return pl.pallas_call(
        _tc1b_body,
        grid=(grid,),
        in_specs=[_row_spec()] + [w, b, w, b, b, b] * 2 + [b, b],
        out_specs=[_row_spec(), _row_spec(), _row_spec()],
        out_shape=[jax.ShapeDtypeStruct((N, H), jnp.float32),
                   jax.ShapeDtypeStruct((N, H), jnp.float32),
                   jax.ShapeDtypeStruct((N, H), jnp.float32)],
    )(x1,
      p['db_W1'].T, p['db_b1'][None], p['db_W2'].T, p['db_b2'][None],
      p['db_g'][None], p['db_b'][None],
      p['rb_W1'].T, p['rb_b1'][None], p['rb_W2'].T, p['rb_b2'][None],
      p['rb_g'][None], p['rb_b'][None],
      p['norm_g'][None], p['norm_b'][None])


def _tc2(seg, alpha, gamma, xres, p):
    grid = N // _ROWBLK
    w = _full_spec((H, H))
    b = _full_spec((1, H))
    return pl.pallas_call(
        _tc2_body,
        grid=(grid,),
        in_specs=[pl.BlockSpec((2, _ROWBLK, H), lambda i: (0, i, 0)),
                  _row_spec(), _row_spec(), _row_spec(), w, b, w, b],
        out_specs=_row_spec(),
        out_shape=jax.ShapeDtypeStruct((N, O), jnp.float32),
    )(seg, alpha, gamma, xres,
      p['fc_W1'].T, p['fc_b1'][None], p['fc_W2'].T, p['fc_b2'][None])


def _sc_body(tab_hbm, gidx_hbm, sidx_hbm, zeros_hbm, out_hbm,
             gv0, sv0, gv1, sv1, gv2, sv2, rows0, rows1, rows2, acc,
             rs0, rs1, rs2, gs0, gs1, gs2, ss0, ss1, ss2):
    cid = lax.axis_index("c")
    sid = lax.axis_index("s")
    pltpu.sync_copy(zeros_hbm.at[pl.ds(sid * ZPT, ZPT)],
                    acc.at[pl.ds(sid * ZPT, ZPT)])
    plsc.subcore_barrier()
    base = cid * EPAD + sid * EPT
    gv = (gv0, gv1, gv2)
    sv = (sv0, sv1, sv2)
    rows = (rows0, rows1, rows2)
    rs = (rs0, rs1, rs2)
    gs = (gs0, gs1, gs2)
    ss = (ss0, ss1, ss2)

    def idx_load(c, p):
        off = base + c * CHUNK
        pltpu.async_copy(gidx_hbm.at[pl.ds(off, CHUNK)], gv[p], gs[p])
        pltpu.async_copy(sidx_hbm.at[pl.ds(off, CHUNK)], sv[p], ss[p])

    def idx_wait(c, p):
        off = base + c * CHUNK
        pltpu.make_async_copy(gidx_hbm.at[pl.ds(off, CHUNK)], gv[p], gs[p]).wait()
        pltpu.make_async_copy(sidx_hbm.at[pl.ds(off, CHUNK)], sv[p], ss[p]).wait()

    def step(c, p):
        q = (p + 2) % 3
        idx_wait(c + 2, q)
        pltpu.async_copy(tab_hbm.at[gv[q]], rows[q], rs[q])
        pltpu.make_async_copy(tab_hbm.at[gv[p]], rows[p], rs[p]).wait()
        pltpu.sync_copy(rows[p], acc.at[sv[p]], add=True)
        idx_load(c + 3, p)

    pltpu.sync_copy(gidx_hbm.at[pl.ds(base, CHUNK)], gv0)
    pltpu.sync_copy(sidx_hbm.at[pl.ds(base, CHUNK)], sv0)
    pltpu.async_copy(tab_hbm.at[gv0], rows0, rs0)
    idx_load(1, 1)
    idx_load(2, 2)
    idx_wait(1, 1)
    pltpu.async_copy(tab_hbm.at[gv1], rows1, rs1)

    def trio(j, carry):
        c = 3 * j
        step(c, 0)
        step(c + 1, 1)
        step(c + 2, 2)
        return carry

    lax.fori_loop(0, (NCHUNK - 2) // 3, trio, 0)
    idx_wait(NCHUNK, 2)
    pltpu.make_async_copy(tab_hbm.at[gv0], rows0, rs0).wait()
    pltpu.sync_copy(rows0, acc.at[sv0], add=True)
    pltpu.make_async_copy(tab_hbm.at[gv1], rows1, rs1).wait()
    pltpu.sync_copy(rows1, acc.at[sv1], add=True)
    plsc.subcore_barrier()
    pltpu.sync_copy(acc.at[pl.ds(sid * ZPT, ZPT)],
                    out_hbm.at[cid, pl.ds(sid * ZPT, ZPT)])


_sc_segsum = functools.partial(
    pl.kernel,
    out_type=jax.ShapeDtypeStruct((2, NROWS, H), jnp.float32),
    mesh=plsc.VectorSubcoreMesh(core_axis_name="c", subcore_axis_name="s",
                                num_cores=NC, num_subcores=NS),
    scratch_types=[
        pltpu.VMEM((CHUNK,), jnp.int32),
        pltpu.VMEM((CHUNK,), jnp.int32),
        pltpu.VMEM((CHUNK,), jnp.int32),
        pltpu.VMEM((CHUNK,), jnp.int32),
        pltpu.VMEM((CHUNK,), jnp.int32),
        pltpu.VMEM((CHUNK,), jnp.int32),
        pltpu.VMEM((CHUNK, H), jnp.float32),
        pltpu.VMEM((CHUNK, H), jnp.float32),
        pltpu.VMEM((CHUNK, H), jnp.float32),
        pltpu.VMEM_SHARED((NROWS, H), jnp.float32),
    ] + [pltpu.SemaphoreType.DMA] * 9,
)(_sc_body)


def kernel(x, edge_index, degree, params):
    del degree
    src = edge_index[0].astype(jnp.int32)
    dst = edge_index[1].astype(jnp.int32)
    pad = EPAD - E
    trash = jnp.full((pad,), N, jnp.int32)
    zpad = jnp.zeros((pad,), jnp.int32)
    tail = jnp.zeros((CHUNK,), jnp.int32)
    gidx = jnp.concatenate([src, zpad, dst + N, zpad, tail])
    sidx = jnp.concatenate([dst, trash, src, trash, tail])

    tab, x1 = _tc1a(x, params)
    seg = _sc_segsum(tab.reshape(2 * N, H), gidx, sidx,
                     jnp.zeros((NROWS, H), jnp.float32))
    alpha, gamma, xres = _tc1b(x1, params)
    return _tc2(seg, alpha, gamma, xres, params)


# speedup vs baseline: 1.3662x; 1.3662x over previous
import functools

import jax
import jax.numpy as jnp
from jax import lax
from jax.experimental import pallas as pl
from jax.experimental.pallas import tpu as pltpu
from jax.experimental.pallas import tpu_sc as plsc

N, E, D, H, O = 10000, 320000, 128, 128, 128

NC, NS = 2, 16
CHUNK = 128
NCHUNK = 157
EPT = NCHUNK * CHUNK
EPAD = NS * EPT
IDXLEN = NC * EPAD + CHUNK
NROWS = 10112
ZPT = NROWS // NS

_ROWBLK = 2000


def _ln(h, g, b):
    m = jnp.mean(h, axis=-1, keepdims=True)
    c = h - m
    v = jnp.mean(c * c, axis=-1, keepdims=True)
    return c / jnp.sqrt(v + 1e-5) * g + b


def _dot(a, b):
    return jax.lax.dot_general(a, b, (((1,), (0,)), ((), ())),
                               preferred_element_type=jnp.float32,
                               precision=jax.lax.Precision.HIGHEST)


def _gelu(x):
    return 0.5 * x * (1.0 + lax.erf(x * 0.7071067811865476))


def _branch(x1, W1, b1, W2, b2, g, b):
    h = _gelu(_dot(x1, W1[...]) + b1[...])
    h = _dot(h, W2[...]) + b2[...]
    return _ln(h, g[...], b[...])


def _tc1a_body(x_ref, linW, linb, rtW1, rtb1, rtW2, rtb2, rtg, rtb,
               tab_o, x1_o):
    x = x_ref[...]
    x1 = _dot(x, linW[...]) + linb[...]
    beta = _branch(x1, rtW1, rtb1, rtW2, rtb2, rtg, rtb)
    tab_o[0, :, :] = x1
    tab_o[1, :, :] = beta * x1
    x1_o[...] = x1


def _tc1b_body(x1_ref, dbW1, dbb1, dbW2, dbb2, dbg, dbb,
               rbW1, rbb1, rbW2, rbb2, rbg, rbb, ng, nb,
               alpha_o, gamma_o, xres_o):
    x1 = x1_ref[...]
    xres_o[...] = _ln(x1, ng[...], nb[...])
    alpha_o[...] = _branch(x1, dbW1, dbb1, dbW2, dbb2, dbg, dbb)
    gamma_o[...] = _branch(x1, rbW1, rbb1, rbW2, rbb2, rbg, rbb)


def _tc2_body(seg_ref, alpha_ref, gamma_ref, xres_ref,
              fcW1, fcb1, fcW2, fcb2, out_o):
    x2 = alpha_ref[...] * seg_ref[0, :, :] + gamma_ref[...] + seg_ref[1, :, :]
    h = _gelu(_dot(x2, fcW1[...]) + fcb1[...])
    out_o[...] = _dot(h, fcW2[...]) + fcb2[...] + xres_ref[...]


def _row_spec():
    return pl.BlockSpec((_ROWBLK, H), lambda i: (i, 0))


def _full_spec(shape):
    return pl.BlockSpec(shape, lambda i: tuple(0 for _ in shape))


def _tc1a(x, p):
    grid = N // _ROWBLK
    w = _full_spec((H, H))
    b = _full_spec((1, H))
    return pl.pallas_call(
        _tc1a_body,
        grid=(grid,),
        in_specs=[_row_spec(), w, b, w, b, w, b, b, b],
        out_specs=[pl.BlockSpec((2, _ROWBLK, H), lambda i: (0, i, 0)),
                   _row_spec()],
        out_shape=[jax.ShapeDtypeStruct((2, N, H), jnp.float32),
                   jax.ShapeDtypeStruct((N, H), jnp.float32)],
    )(x,
      p['lin_W'].T, p['lin_b'][None],
      p['rt_W1'].T, p['rt_b1'][None], p['rt_W2'].T, p['rt_b2'][None],
      p['rt_g'][None], p['rt_b'][None])


def _tc1b(x1, p):
    grid = N // _ROWBLK
    w = _full_spec((H, H))
    b = _full_spec((1, H))
    return pl.pallas_call(
        _tc1b_body,
        grid=(grid,),
        in_specs=[_row_spec()] + [w, b, w, b, b, b] * 2 + [b, b],
        out_specs=[_row_spec(), _row_spec(), _row_spec()],
        out_shape=[jax.ShapeDtypeStruct((N, H), jnp.float32),
                   jax.ShapeDtypeStruct((N, H), jnp.float32),
                   jax.ShapeDtypeStruct((N, H), jnp.float32)],
    )(x1,
      p['db_W1'].T, p['db_b1'][None], p['db_W2'].T, p['db_b2'][None],
      p['db_g'][None], p['db_b'][None],
      p['rb_W1'].T, p['rb_b1'][None], p['rb_W2'].T, p['rb_b2'][None],
      p['rb_g'][None], p['rb_b'][None],
      p['norm_g'][None], p['norm_b'][None])


def _tc2(seg, alpha, gamma, xres, p):
    grid = N // _ROWBLK
    w = _full_spec((H, H))
    b = _full_spec((1, H))
    return pl.pallas_call(
        _tc2_body,
        grid=(grid,),
        in_specs=[pl.BlockSpec((2, _ROWBLK, H), lambda i: (0, i, 0)),
                  _row_spec(), _row_spec(), _row_spec(), w, b, w, b],
        out_specs=_row_spec(),
        out_shape=jax.ShapeDtypeStruct((N, O), jnp.float32),
    )(seg, alpha, gamma, xres,
      p['fc_W1'].T, p['fc_b1'][None], p['fc_W2'].T, p['fc_b2'][None])


def _sc_body(tab_hbm, gidx_hbm, sidx_hbm, zeros_hbm, out_hbm,
             gv0, sv0, gv1, sv1, rows0, rows1, acc,
             rs0, rs1, gs0, ss0, gs1, ss1):
    cid = lax.axis_index("c")
    sid = lax.axis_index("s")
    pltpu.sync_copy(zeros_hbm.at[pl.ds(sid * ZPT, ZPT)],
                    acc.at[pl.ds(sid * ZPT, ZPT)])
    plsc.subcore_barrier()
    base = cid * EPAD + sid * EPT

    def idx_load(off, gv, sv, gs, ss):
        pltpu.async_copy(gidx_hbm.at[pl.ds(off, CHUNK)], gv, gs)
        pltpu.async_copy(sidx_hbm.at[pl.ds(off, CHUNK)], sv, ss)

    def idx_wait(off, gv, sv, gs, ss):
        pltpu.make_async_copy(gidx_hbm.at[pl.ds(off, CHUNK)], gv, gs).wait()
        pltpu.make_async_copy(sidx_hbm.at[pl.ds(off, CHUNK)], sv, ss).wait()

    pltpu.sync_copy(gidx_hbm.at[pl.ds(base, CHUNK)], gv0)
    pltpu.sync_copy(sidx_hbm.at[pl.ds(base, CHUNK)], sv0)
    pltpu.async_copy(tab_hbm.at[gv0], rows0, rs0)
    idx_load(base + CHUNK, gv1, sv1, gs1, ss1)

    def pair(j, carry):
        offb = base + (2 * j + 1) * CHUNK
        offa2 = offb + CHUNK
        offb2 = offa2 + CHUNK
        idx_wait(offb, gv1, sv1, gs1, ss1)
        pltpu.async_copy(tab_hbm.at[gv1], rows1, rs1)
        pltpu.make_async_copy(tab_hbm.at[gv0], rows0, rs0).wait()
        pltpu.sync_copy(rows0, acc.at[sv0], add=True)
        idx_load(offa2, gv0, sv0, gs0, ss0)
        idx_wait(offa2, gv0, sv0, gs0, ss0)
        pltpu.async_copy(tab_hbm.at[gv0], rows0, rs0)
        pltpu.make_async_copy(tab_hbm.at[gv1], rows1, rs1).wait()
        pltpu.sync_copy(rows1, acc.at[sv1], add=True)
        idx_load(offb2, gv1, sv1, gs1, ss1)
        return carry

    lax.fori_loop(0, NCHUNK // 2, pair, 0)
    idx_wait(base + NCHUNK * CHUNK, gv1, sv1, gs1, ss1)
    pltpu.make_async_copy(tab_hbm.at[gv0], rows0, rs0).wait()
    pltpu.sync_copy(rows0, acc.at[sv0], add=True)
    plsc.subcore_barrier()
    pltpu.sync_copy(acc.at[pl.ds(sid * ZPT, ZPT)],
                    out_hbm.at[cid, pl.ds(sid * ZPT, ZPT)])


_sc_segsum = functools.partial(
    pl.kernel,
    out_type=jax.ShapeDtypeStruct((2, NROWS, H), jnp.float32),
    mesh=plsc.VectorSubcoreMesh(core_axis_name="c", subcore_axis_name="s",
                                num_cores=NC, num_subcores=NS),
    scratch_types=[
        pltpu.VMEM((CHUNK,), jnp.int32),
        pltpu.VMEM((CHUNK,), jnp.int32),
        pltpu.VMEM((CHUNK,), jnp.int32),
        pltpu.VMEM((CHUNK,), jnp.int32),
        pltpu.VMEM((CHUNK, H), jnp.float32),
        pltpu.VMEM((CHUNK, H), jnp.float32),
        pltpu.VMEM_SHARED((NROWS, H), jnp.float32),
    ] + [pltpu.SemaphoreType.DMA] * 6,
)(_sc_body)


def kernel(x, edge_index, degree, params):
    del degree
    src = edge_index[0].astype(jnp.int32)
    dst = edge_index[1].astype(jnp.int32)
    pad = EPAD - E
    trash = jnp.full((pad,), N, jnp.int32)
    zpad = jnp.zeros((pad,), jnp.int32)
    tail = jnp.zeros((CHUNK,), jnp.int32)
    gidx = jnp.concatenate([src, zpad, dst + N, zpad, tail])
    sidx = jnp.concatenate([dst, trash, src, trash, tail])

    tab, x1 = _tc1a(x, params)
    seg = _sc_segsum(tab.reshape(2 * N, H), gidx, sidx,
                     jnp.zeros((NROWS, H), jnp.float32))
    alpha, gamma, xres = _tc1b(x1, params)
    return _tc2(seg, alpha, gamma, xres, params)
